# baseline (speedup 1.0000x reference)
import jax, jax.numpy as jnp
from jax.experimental import pallas as pl

def kernel(x, edge_index, node_norm, edge_norm, W, b):
    n = x.shape[0]
    src = edge_index[0]; dst = edge_index[1]
    norm = node_norm[src] * node_norm[dst] * edge_norm
    s = jax.ops.segment_sum(norm, dst, num_segments=n)
    sb = s[:, None] * b[None, :]
    h = x
    outs = [jnp.max(h, axis=0, keepdims=True)]
    for _ in range(2):
        hw = jnp.dot(h, W, precision=jax.lax.Precision.HIGHEST)
        a = jax.ops.segment_sum(norm[:, None] * hw[src], dst, num_segments=n)
        h = jax.nn.leaky_relu(a + sb, negative_slope=0.01)
        outs.append(jnp.max(h, axis=0, keepdims=True))
    return jnp.mean(jnp.stack(outs, axis=0), axis=0)


# speedup vs baseline: 1.5843x; 1.5843x over previous
import functools

import jax
import jax.numpy as jnp
from jax import lax
from jax.experimental import pallas as pl
from jax.experimental.pallas import tpu as pltpu
from jax.experimental.pallas import tpu_sc as plsc

_NS = 16
_NC = 2
_CHUNK = 80
_QCOL = 64


def _agg_kernel_body(nchunk, rpt, hw4, src_t, dst_t, norm_t, zrows, out,
                     idx_v, dst_v, norm_v, rows_v, acc, gsem):
    cid = lax.axis_index("c")
    sid = lax.axis_index("s")

    pltpu.sync_copy(src_t.at[sid], idx_v)
    pltpu.sync_copy(dst_t.at[sid], dst_v)
    pltpu.sync_copy(norm_t.at[sid], norm_v)

    @pl.loop(0, nchunk)
    def _scale_idx(j):
        for q in range(_CHUNK // 16):
            sl = (j, pl.ds(q * 16, 16))
            idx_v[sl] = idx_v[sl] * 4 + 2 * cid

    def _start_gather(j, buf):
        pltpu.async_copy(hw4.at[idx_v.at[j]], rows_v.at[buf], gsem)

    def _wait_gather(j, buf):
        pltpu.make_async_copy(hw4.at[idx_v.at[j]], rows_v.at[buf], gsem).wait()

    for p in range(2):
        if p == 1:
            @pl.loop(0, nchunk)
            def _bump_idx(j):
                for q in range(_CHUNK // 16):
                    sl = (j, pl.ds(q * 16, 16))
                    idx_v[sl] = idx_v[sl] + 1

        pltpu.sync_copy(zrows, acc.at[pl.ds(sid * rpt, rpt)])
        plsc.subcore_barrier()

        _start_gather(0, 0)

        @pl.loop(0, nchunk, step=2)
        def _edge_chunks(j):
            for t in range(2):
                cur = t
                jj = j + t

                _wait_gather(jj, cur)

                @pl.when(jj + 1 < nchunk)
                def _():
                    _start_gather(jj + 1, 1 - t)

                @pl.loop(0, _CHUNK // 16)
                def _scale_rows(k):
                    nvv = norm_v[jj, pl.ds(k * 16, 16)]
                    for i in range(16):
                        nv = nvv[i]
                        r = k * 16 + i
                        for q in range(_QCOL // 16):
                            sl = (cur, r, pl.ds(q * 16, 16))
                            rows_v[sl] = rows_v[sl] * nv

                pltpu.sync_copy(rows_v.at[cur], acc.at[dst_v.at[jj]], add=True)

        plsc.subcore_barrier()
        pltpu.sync_copy(acc.at[pl.ds(sid * rpt, rpt)],
                        out.at[2 * cid + p].at[pl.ds(sid * rpt, rpt)])


def _sc_aggregate(hw4, src_t, dst_t, norm_t, zrows, n_pad):
    nchunk = src_t.shape[1]
    rpt = n_pad // _NS
    mesh = plsc.VectorSubcoreMesh(core_axis_name="c", subcore_axis_name="s")
    body = functools.partial(_agg_kernel_body, nchunk, rpt)
    return pl.kernel(
        body,
        out_type=jax.ShapeDtypeStruct((4, n_pad, _QCOL), jnp.float32),
        mesh=mesh,
        compiler_params=pltpu.CompilerParams(use_tc_tiling_on_sc=False),
        scratch_types=[
            pltpu.VMEM((nchunk, _CHUNK), jnp.int32),
            pltpu.VMEM((nchunk, _CHUNK), jnp.int32),
            pltpu.VMEM((nchunk, _CHUNK), jnp.float32),
            pltpu.VMEM((2, _CHUNK, _QCOL), jnp.float32),
            pltpu.VMEM_SHARED((n_pad, _QCOL), jnp.float32),
            pltpu.SemaphoreType.DMA,
        ],
    )(hw4, src_t, dst_t, norm_t, zrows)


def kernel(x, edge_index, node_norm, edge_norm, W, b):
    n, d = x.shape
    e = edge_index.shape[1]
    src = edge_index[0]
    dst = edge_index[1]

    quant = 2560
    e_pad = ((e + quant - 1) // quant) * quant
    pad = e_pad - e
    src_p = jnp.pad(src, (0, pad))
    dst_p = jnp.pad(dst, (0, pad))
    en_p = jnp.pad(edge_norm, (0, pad))

    norm = node_norm[src_p] * node_norm[dst_p] * en_p
    s = jax.ops.segment_sum(norm, dst_p, num_segments=n)
    sb = s[:, None] * b[None, :]

    nchunk = e_pad // (_NS * _CHUNK)
    src_t = src_p.reshape(_NS, nchunk, _CHUNK)
    dst_t = dst_p.reshape(_NS, nchunk, _CHUNK)
    norm_t = norm.reshape(_NS, nchunk, _CHUNK)
    n_pad = ((n + _NS * 8 - 1) // (_NS * 8)) * (_NS * 8)
    zrows = jnp.zeros((n_pad // _NS, _QCOL), jnp.float32)

    h = x
    outs = [jnp.max(h, axis=0, keepdims=True)]
    for _ in range(2):
        hw = jnp.dot(h, W, precision=lax.Precision.HIGHEST)
        hw4 = hw.reshape(4 * n, _QCOL)
        a4 = _sc_aggregate(hw4, src_t, dst_t, norm_t, zrows, n_pad)
        a = jnp.concatenate([a4[0, :n], a4[1, :n], a4[2, :n], a4[3, :n]],
                            axis=1)
        h = jax.nn.leaky_relu(a + sb, negative_slope=0.01)
        outs.append(jnp.max(h, axis=0, keepdims=True))
    return jnp.mean(jnp.stack(outs, axis=0), axis=0)


# speedup vs baseline: 4.0058x; 2.5284x over previous
import functools
import math

import jax
import jax.numpy as jnp
from jax import lax
from jax.experimental import pallas as pl
from jax.experimental.pallas import tpu as pltpu
from jax.experimental.pallas import tpu_sc as plsc

_NS = 16
_NC = 2
_CHUNK = 128
_QCOL = 64
_NBUF = 3

_SC_PARAMS = pltpu.CompilerParams(use_tc_tiling_on_sc=False,
                                  needs_layout_passes=False)
_SC_PARAMS_AGG = pltpu.CompilerParams(use_tc_tiling_on_sc=False)


def _norm_kernel_body(n_pad, ept, node_norm, src, dst, en, norm_out, sparts,
                      nn_v, sv_v, dv_v, ev_v, no_v, sp_v):
    cid = lax.axis_index("c")
    sid = lax.axis_index("s")
    w = sid * _NC + cid

    pltpu.sync_copy(node_norm, nn_v)
    base = w * ept
    pltpu.sync_copy(src.at[pl.ds(base, ept)], sv_v)
    pltpu.sync_copy(dst.at[pl.ds(base, ept)], dv_v)
    pltpu.sync_copy(en.at[pl.ds(base, ept)], ev_v)

    zero16 = jnp.zeros((16,), jnp.float32)

    @pl.loop(0, n_pad // 16)
    def _zero(i):
        sp_v[pl.ds(i * 16, 16)] = zero16

    @pl.loop(0, ept // 16)
    def _edges(i):
        sl = pl.ds(i * 16, 16)
        sv = sv_v[sl]
        dv = dv_v[sl]
        ev = ev_v[sl]
        a = plsc.load_gather(nn_v, [sv])
        c = plsc.load_gather(nn_v, [dv])
        nr = a * c * ev
        no_v[sl] = nr
        plsc.addupdate_scatter(sp_v, [dv], nr)

    pltpu.sync_copy(no_v, norm_out.at[pl.ds(base, ept)])
    pltpu.sync_copy(sp_v, sparts.at[w])


def _sc_edge_norms(node_norm, src_p, dst_p, en_p, n_pad):
    e_pad = src_p.shape[0]
    ept = e_pad // (_NC * _NS)
    mesh = plsc.VectorSubcoreMesh(core_axis_name="c", subcore_axis_name="s")
    body = functools.partial(_norm_kernel_body, n_pad, ept)
    return pl.kernel(
        body,
        out_type=(jax.ShapeDtypeStruct((e_pad,), jnp.float32),
                  jax.ShapeDtypeStruct((_NC * _NS, n_pad), jnp.float32)),
        mesh=mesh,
        compiler_params=_SC_PARAMS,
        scratch_types=[
            pltpu.VMEM(node_norm.shape, jnp.float32),
            pltpu.VMEM((ept,), jnp.int32),
            pltpu.VMEM((ept,), jnp.int32),
            pltpu.VMEM((ept,), jnp.float32),
            pltpu.VMEM((ept,), jnp.float32),
            pltpu.VMEM((n_pad,), jnp.float32),
        ],
    )(node_norm, src_p, dst_p, en_p)


def _agg_kernel_body(nchunk, rpt, hw4, src_t, dst_t, norm_t, zrows, out,
                     idx_v, dst_v, norm_v, rows_v, acc, gsem, ssem):
    cid = lax.axis_index("c")
    sid = lax.axis_index("s")

    pltpu.sync_copy(src_t.at[sid], idx_v)
    pltpu.sync_copy(dst_t.at[sid], dst_v)
    pltpu.sync_copy(norm_t.at[sid], norm_v)

    @pl.loop(0, nchunk)
    def _scale_idx(j):
        for q in range(_CHUNK // 16):
            sl = (j, pl.ds(q * 16, 16))
            idx_v[sl] = idx_v[sl] * 4 + 2 * cid

    def _gather(j, buf):
        pltpu.async_copy(hw4.at[idx_v.at[j]], rows_v.at[buf], gsem)

    def _wait_gather(j, buf):
        pltpu.make_async_copy(hw4.at[idx_v.at[j]], rows_v.at[buf], gsem).wait()

    def _scatter(j, buf):
        pltpu.async_copy(rows_v.at[buf], acc.at[dst_v.at[j]], ssem, add=True)

    def _wait_scatter(j, buf):
        pltpu.make_async_copy(rows_v.at[buf], acc.at[dst_v.at[j]], ssem).wait()

    for p in range(2):
        if p == 1:
            @pl.loop(0, nchunk)
            def _bump_idx(j):
                for q in range(_CHUNK // 16):
                    sl = (j, pl.ds(q * 16, 16))
                    idx_v[sl] = idx_v[sl] + 1

        pltpu.sync_copy(zrows, acc.at[pl.ds(sid * rpt, rpt)])
        plsc.subcore_barrier()

        _gather(0, 0)

        @pl.loop(0, nchunk, step=2)
        def _edge_chunks(j):
            for t in range(2):
                buf = t
                jj = j + t

                _wait_gather(jj, buf)

                @pl.when(jj + 1 < nchunk)
                def _():
                    _gather(jj + 1, 1 - t)

                @pl.loop(0, _CHUNK // 16)
                def _scale_rows(k):
                    nvv = norm_v[jj, pl.ds(k * 16, 16)]
                    for i in range(16):
                        nv = nvv[i]
                        r = k * 16 + i
                        for q in range(_QCOL // 16):
                            sl = (buf, r, pl.ds(q * 16, 16))
                            rows_v[sl] = rows_v[sl] * nv

                pltpu.sync_copy(rows_v.at[buf], acc.at[dst_v.at[jj]],
                                add=True)

        plsc.subcore_barrier()
        pltpu.sync_copy(acc.at[pl.ds(sid * rpt, rpt)],
                        out.at[2 * cid + p].at[pl.ds(sid * rpt, rpt)])


def _sc_aggregate(hw4, src_t, dst_t, norm_t, zrows, n_pad):
    nchunk = src_t.shape[1]
    rpt = n_pad // _NS
    mesh = plsc.VectorSubcoreMesh(core_axis_name="c", subcore_axis_name="s")
    body = functools.partial(_agg_kernel_body, nchunk, rpt)
    return pl.kernel(
        body,
        out_type=jax.ShapeDtypeStruct((4, n_pad, _QCOL), jnp.float32),
        mesh=mesh,
        compiler_params=_SC_PARAMS_AGG,
        scratch_types=[
            pltpu.VMEM((nchunk, _CHUNK), jnp.int32),
            pltpu.VMEM((nchunk, _CHUNK), jnp.int32),
            pltpu.VMEM((nchunk, _CHUNK), jnp.float32),
            pltpu.VMEM((_NBUF, _CHUNK, _QCOL), jnp.float32),
            pltpu.VMEM_SHARED((n_pad, _QCOL), jnp.float32),
            pltpu.SemaphoreType.DMA,
            pltpu.SemaphoreType.DMA,
        ],
    )(hw4, src_t, dst_t, norm_t, zrows)


def kernel(x, edge_index, node_norm, edge_norm, W, b):
    n, d = x.shape
    e = edge_index.shape[1]
    src = edge_index[0]
    dst = edge_index[1]

    quant = math.lcm(_NS * _CHUNK * 2, _NC * _NS * 16)
    e_pad = ((e + quant - 1) // quant) * quant
    pad = e_pad - e
    src_p = jnp.pad(src, (0, pad))
    dst_p = jnp.pad(dst, (0, pad))
    en_p = jnp.pad(edge_norm, (0, pad))

    n_pad = ((n + _NS * 8 - 1) // (_NS * 8)) * (_NS * 8)

    norm, sparts = _sc_edge_norms(node_norm, src_p, dst_p, en_p, n_pad)
    s = jnp.sum(sparts, axis=0)[:n]
    sb = s[:, None] * b[None, :]

    nchunk = e_pad // (_NS * _CHUNK)
    src_t = src_p.reshape(_NS, nchunk, _CHUNK)
    dst_t = dst_p.reshape(_NS, nchunk, _CHUNK)
    norm_t = norm.reshape(_NS, nchunk, _CHUNK)
    zrows = jnp.zeros((n_pad // _NS, _QCOL), jnp.float32)

    h = x
    outs = [jnp.max(h, axis=0, keepdims=True)]
    for _ in range(2):
        hw = jnp.dot(h, W, precision=lax.Precision.HIGHEST)
        hw4 = hw.reshape(4 * n, _QCOL)
        a4 = _sc_aggregate(hw4, src_t, dst_t, norm_t, zrows, n_pad)
        a = jnp.concatenate([a4[0, :n], a4[1, :n], a4[2, :n], a4[3, :n]],
                            axis=1)
        h = jax.nn.leaky_relu(a + sb, negative_slope=0.01)
        outs.append(jnp.max(h, axis=0, keepdims=True))
    return jnp.mean(jnp.stack(outs, axis=0), axis=0)


# speedup vs baseline: 5.0558x; 1.2621x over previous
import functools
import math

import jax
import jax.numpy as jnp
from jax import lax
from jax.experimental import pallas as pl
from jax.experimental.pallas import tpu as pltpu
from jax.experimental.pallas import tpu_sc as plsc

_NS = 16
_NC = 2
_CHUNK = 128
_QCOL = 64
_GRP = 2

_SC_PARAMS = pltpu.CompilerParams(use_tc_tiling_on_sc=False,
                                  needs_layout_passes=False)
_SC_PARAMS_AGG = pltpu.CompilerParams(use_tc_tiling_on_sc=False)


def _norm_kernel_body(n_pad, ept, node_norm, src, dst, en, norm_out, sparts,
                      nn_v, sv_v, dv_v, ev_v, no_v, sp_v):
    cid = lax.axis_index("c")
    sid = lax.axis_index("s")
    w = sid * _NC + cid

    pltpu.sync_copy(node_norm, nn_v)
    base = w * ept
    pltpu.sync_copy(src.at[pl.ds(base, ept)], sv_v)
    pltpu.sync_copy(dst.at[pl.ds(base, ept)], dv_v)
    pltpu.sync_copy(en.at[pl.ds(base, ept)], ev_v)

    zero16 = jnp.zeros((16,), jnp.float32)

    @pl.loop(0, n_pad // 16)
    def _zero(i):
        sp_v[pl.ds(i * 16, 16)] = zero16

    @pl.loop(0, ept // 16)
    def _edges(i):
        sl = pl.ds(i * 16, 16)
        sv = sv_v[sl]
        dv = dv_v[sl]
        ev = ev_v[sl]
        a = plsc.load_gather(nn_v, [sv])
        c = plsc.load_gather(nn_v, [dv])
        nr = a * c * ev
        no_v[sl] = nr
        plsc.addupdate_scatter(sp_v, [dv], nr)

    pltpu.sync_copy(no_v, norm_out.at[pl.ds(base, ept)])
    pltpu.sync_copy(sp_v, sparts.at[w])


def _sc_edge_norms(node_norm, src_p, dst_p, en_p, n_pad):
    e_pad = src_p.shape[0]
    ept = e_pad // (_NC * _NS)
    mesh = plsc.VectorSubcoreMesh(core_axis_name="c", subcore_axis_name="s")
    body = functools.partial(_norm_kernel_body, n_pad, ept)
    return pl.kernel(
        body,
        out_type=(jax.ShapeDtypeStruct((e_pad,), jnp.float32),
                  jax.ShapeDtypeStruct((_NC * _NS, n_pad), jnp.float32)),
        mesh=mesh,
        compiler_params=_SC_PARAMS,
        scratch_types=[
            pltpu.VMEM(node_norm.shape, jnp.float32),
            pltpu.VMEM((ept,), jnp.int32),
            pltpu.VMEM((ept,), jnp.int32),
            pltpu.VMEM((ept,), jnp.float32),
            pltpu.VMEM((ept,), jnp.float32),
            pltpu.VMEM((n_pad,), jnp.float32),
        ],
    )(node_norm, src_p, dst_p, en_p)


def _agg_kernel_body(nchunk, rpt, hw4, src_t, dst_t, norm_t, zrows, out,
                     idx_v, dst_v, norm_v, rows_v, acc, gsem, ssem):
    cid = lax.axis_index("c")
    sid = lax.axis_index("s")

    pltpu.sync_copy(src_t.at[sid], idx_v)
    pltpu.sync_copy(dst_t.at[sid], dst_v)
    pltpu.sync_copy(norm_t.at[sid], norm_v)

    @pl.loop(0, nchunk)
    def _scale_idx(j):
        for q in range(_CHUNK // 16):
            sl = (j, pl.ds(q * 16, 16))
            idx_v[sl] = idx_v[sl] * 4 + 2 * cid

    def _gather(j, buf):
        pltpu.async_copy(hw4.at[idx_v.at[j]], rows_v.at[buf], gsem)

    def _wait_gather(j, buf):
        pltpu.make_async_copy(hw4.at[idx_v.at[j]], rows_v.at[buf], gsem).wait()

    def _scale(jj, buf):
        @pl.loop(0, _CHUNK // 16)
        def _scale_rows(k):
            nvv = norm_v[jj, pl.ds(k * 16, 16)]
            for i in range(16):
                nv = nvv[i]
                r = k * 16 + i
                for q in range(_QCOL // 16):
                    sl = (buf, r, pl.ds(q * 16, 16))
                    rows_v[sl] = rows_v[sl] * nv

    for p in range(2):
        if p == 1:
            @pl.loop(0, nchunk)
            def _bump_idx(j):
                for q in range(_CHUNK // 16):
                    sl = (j, pl.ds(q * 16, 16))
                    idx_v[sl] = idx_v[sl] + 1

        pltpu.sync_copy(zrows, acc.at[pl.ds(sid * rpt, rpt)])
        plsc.subcore_barrier()

        for t in range(2 * _GRP):
            _gather(t, t)

        @pl.loop(0, nchunk, step=2 * _GRP)
        def _edge_chunks(j):
            for half in range(2):
                descs = []
                for t in range(_GRP):
                    buf = half * _GRP + t
                    jj = j + buf
                    _wait_gather(jj, buf)
                    _scale(jj, buf)
                    descs.append(pltpu.async_copy(
                        rows_v.at[buf], acc.at[dst_v.at[jj]], ssem, add=True))
                for t in range(_GRP):
                    buf = half * _GRP + t
                    jj = j + buf
                    descs[t].wait()

                    @pl.when(jj + 2 * _GRP < nchunk)
                    def _():
                        _gather(jj + 2 * _GRP, buf)

        plsc.subcore_barrier()
        pltpu.sync_copy(acc.at[pl.ds(sid * rpt, rpt)],
                        out.at[2 * cid + p].at[pl.ds(sid * rpt, rpt)])


def _sc_aggregate(hw4, src_t, dst_t, norm_t, zrows, n_pad):
    nchunk = src_t.shape[1]
    rpt = n_pad // _NS
    mesh = plsc.VectorSubcoreMesh(core_axis_name="c", subcore_axis_name="s")
    body = functools.partial(_agg_kernel_body, nchunk, rpt)
    return pl.kernel(
        body,
        out_type=jax.ShapeDtypeStruct((4, n_pad, _QCOL), jnp.float32),
        mesh=mesh,
        compiler_params=_SC_PARAMS_AGG,
        scratch_types=[
            pltpu.VMEM((nchunk, _CHUNK), jnp.int32),
            pltpu.VMEM((nchunk, _CHUNK), jnp.int32),
            pltpu.VMEM((nchunk, _CHUNK), jnp.float32),
            pltpu.VMEM((2 * _GRP, _CHUNK, _QCOL), jnp.float32),
            pltpu.VMEM_SHARED((n_pad, _QCOL), jnp.float32),
            pltpu.SemaphoreType.DMA,
            pltpu.SemaphoreType.DMA,
        ],
    )(hw4, src_t, dst_t, norm_t, zrows)


def kernel(x, edge_index, node_norm, edge_norm, W, b):
    n, d = x.shape
    e = edge_index.shape[1]
    src = edge_index[0]
    dst = edge_index[1]

    quant = math.lcm(_NS * _CHUNK * 2 * _GRP, _NC * _NS * 16)
    e_pad = ((e + quant - 1) // quant) * quant
    pad = e_pad - e
    src_p = jnp.pad(src, (0, pad))
    dst_p = jnp.pad(dst, (0, pad))
    en_p = jnp.pad(edge_norm, (0, pad))

    n_pad = ((n + _NS * 8 - 1) // (_NS * 8)) * (_NS * 8)

    norm, sparts = _sc_edge_norms(node_norm, src_p, dst_p, en_p, n_pad)
    s = jnp.sum(sparts, axis=0)[:n]
    sb = s[:, None] * b[None, :]

    nchunk = e_pad // (_NS * _CHUNK)
    src_t = src_p.reshape(_NS, nchunk, _CHUNK)
    dst_t = dst_p.reshape(_NS, nchunk, _CHUNK)
    norm_t = norm.reshape(_NS, nchunk, _CHUNK)
    zrows = jnp.zeros((n_pad // _NS, _QCOL), jnp.float32)

    h = x
    outs = [jnp.max(h, axis=0, keepdims=True)]
    for _ in range(2):
        hw = jnp.dot(h, W, precision=lax.Precision.HIGHEST)
        hw4 = hw.reshape(4 * n, _QCOL)
        a4 = _sc_aggregate(hw4, src_t, dst_t, norm_t, zrows, n_pad)
        a = jnp.concatenate([a4[0, :n], a4[1, :n], a4[2, :n], a4[3, :n]],
                            axis=1)
        h = jax.nn.leaky_relu(a + sb, negative_slope=0.01)
        outs.append(jnp.max(h, axis=0, keepdims=True))
    return jnp.mean(jnp.stack(outs, axis=0), axis=0)


# speedup vs baseline: 5.3691x; 1.0620x over previous
import functools
import math

import jax
import jax.numpy as jnp
from jax import lax
from jax.experimental import pallas as pl
from jax.experimental.pallas import tpu as pltpu
from jax.experimental.pallas import tpu_sc as plsc

_NS = 16
_NC = 2
_CHUNK = 128
_QCOL = 64
_GRP = 2

_SC_PARAMS = pltpu.CompilerParams(use_tc_tiling_on_sc=False,
                                  needs_layout_passes=False)
_SC_PARAMS_AGG = pltpu.CompilerParams(use_tc_tiling_on_sc=False)


def _norm_kernel_body(n_pad, ept, node_norm, src, dst, en, norm_out, sparts,
                      nn_v, sv_v, dv_v, ev_v, no_v, sp_v):
    cid = lax.axis_index("c")
    sid = lax.axis_index("s")
    w = sid * _NC + cid

    pltpu.sync_copy(node_norm, nn_v)
    base = w * ept
    pltpu.sync_copy(src.at[pl.ds(base, ept)], sv_v)
    pltpu.sync_copy(dst.at[pl.ds(base, ept)], dv_v)
    pltpu.sync_copy(en.at[pl.ds(base, ept)], ev_v)

    zero16 = jnp.zeros((16,), jnp.float32)

    @pl.loop(0, n_pad // 16)
    def _zero(i):
        sp_v[pl.ds(i * 16, 16)] = zero16

    @pl.loop(0, ept // 16)
    def _edges(i):
        sl = pl.ds(i * 16, 16)
        sv = sv_v[sl]
        dv = dv_v[sl]
        ev = ev_v[sl]
        a = plsc.load_gather(nn_v, [sv])
        c = plsc.load_gather(nn_v, [dv])
        nr = a * c * ev
        no_v[sl] = nr
        plsc.addupdate_scatter(sp_v, [dv], nr)

    pltpu.sync_copy(no_v, norm_out.at[pl.ds(base, ept)])
    pltpu.sync_copy(sp_v, sparts.at[w])


def _sc_edge_norms(node_norm, src_p, dst_p, en_p, n_pad):
    e_pad = src_p.shape[0]
    ept = e_pad // (_NC * _NS)
    mesh = plsc.VectorSubcoreMesh(core_axis_name="c", subcore_axis_name="s")
    body = functools.partial(_norm_kernel_body, n_pad, ept)
    return pl.kernel(
        body,
        out_type=(jax.ShapeDtypeStruct((e_pad,), jnp.float32),
                  jax.ShapeDtypeStruct((_NC * _NS, n_pad), jnp.float32)),
        mesh=mesh,
        compiler_params=_SC_PARAMS,
        scratch_types=[
            pltpu.VMEM(node_norm.shape, jnp.float32),
            pltpu.VMEM((ept,), jnp.int32),
            pltpu.VMEM((ept,), jnp.int32),
            pltpu.VMEM((ept,), jnp.float32),
            pltpu.VMEM((ept,), jnp.float32),
            pltpu.VMEM((n_pad,), jnp.float32),
        ],
    )(node_norm, src_p, dst_p, en_p)


def _agg_kernel_body(nchunk, rpt, hw4, src_t, dst_t, norm_t, zrows, out,
                     idx_v, dst_v, norm_v, rows_v, acc, gsem, ssem):
    cid = lax.axis_index("c")
    sid = lax.axis_index("s")

    pltpu.sync_copy(src_t.at[sid], idx_v)
    pltpu.sync_copy(dst_t.at[sid], dst_v)
    pltpu.sync_copy(norm_t.at[sid], norm_v)

    @pl.loop(0, nchunk)
    def _scale_idx(j):
        for q in range(_CHUNK // 16):
            sl = (j, pl.ds(q * 16, 16))
            idx_v[sl] = idx_v[sl] * 4 + 2 * cid

    def _gather(j, buf):
        pltpu.async_copy(hw4.at[idx_v.at[j]], rows_v.at[buf], gsem)

    def _wait_gather(j, buf):
        pltpu.make_async_copy(hw4.at[idx_v.at[j]], rows_v.at[buf], gsem).wait()

    def _scale(jj, buf):
        @pl.loop(0, _CHUNK // 16)
        def _scale_rows(k):
            nvv = norm_v[jj, pl.ds(k * 16, 16)]
            for i in range(16):
                nv = nvv[i]
                r = k * 16 + i
                for q in range(_QCOL // 16):
                    sl = (buf, r, pl.ds(q * 16, 16))
                    rows_v[sl] = rows_v[sl] * nv

    for p in range(2):
        if p == 1:
            @pl.loop(0, nchunk)
            def _bump_idx(j):
                for q in range(_CHUNK // 16):
                    sl = (j, pl.ds(q * 16, 16))
                    idx_v[sl] = idx_v[sl] + 1

        pltpu.sync_copy(zrows, acc.at[pl.ds(sid * rpt, rpt)])
        plsc.subcore_barrier()

        for t in range(2 * _GRP):
            _gather(t, t)

        @pl.loop(0, nchunk, step=2 * _GRP)
        def _edge_chunks(j):
            for half in range(2):
                descs = []
                for t in range(_GRP):
                    buf = half * _GRP + t
                    jj = j + buf
                    _wait_gather(jj, buf)
                    _scale(jj, buf)
                    descs.append(pltpu.async_copy(
                        rows_v.at[buf], acc.at[dst_v.at[jj]], ssem, add=True))
                for t in range(_GRP):
                    buf = half * _GRP + t
                    jj = j + buf
                    descs[t].wait()

                    @pl.when(jj + 2 * _GRP < nchunk)
                    def _():
                        _gather(jj + 2 * _GRP, buf)

        plsc.subcore_barrier()
        pltpu.sync_copy(acc.at[pl.ds(sid * rpt, rpt)],
                        out.at[2 * cid + p].at[pl.ds(sid * rpt, rpt)])


def _sc_aggregate(hw4, src_t, dst_t, norm_t, zrows, n_pad):
    nchunk = src_t.shape[1]
    rpt = n_pad // _NS
    mesh = plsc.VectorSubcoreMesh(core_axis_name="c", subcore_axis_name="s")
    body = functools.partial(_agg_kernel_body, nchunk, rpt)
    return pl.kernel(
        body,
        out_type=jax.ShapeDtypeStruct((4, n_pad, _QCOL), jnp.float32),
        mesh=mesh,
        compiler_params=_SC_PARAMS_AGG,
        scratch_types=[
            pltpu.VMEM((nchunk, _CHUNK), jnp.int32),
            pltpu.VMEM((nchunk, _CHUNK), jnp.int32),
            pltpu.VMEM((nchunk, _CHUNK), jnp.float32),
            pltpu.VMEM((2 * _GRP, _CHUNK, _QCOL), jnp.float32),
            pltpu.VMEM_SHARED((n_pad, _QCOL), jnp.float32),
            pltpu.SemaphoreType.DMA,
            pltpu.SemaphoreType.DMA,
        ],
    )(hw4, src_t, dst_t, norm_t, zrows)


_BLK = 2000


def _tc_stage0_body(x_ref, w_ref, hw_ref, mx_ref):
    i = pl.program_id(0)
    xb = x_ref[...]
    hw_ref[...] = jnp.dot(xb, w_ref[...], precision=lax.Precision.HIGHEST)
    m = jnp.max(xb, axis=0, keepdims=True)

    @pl.when(i == 0)
    def _():
        mx_ref[...] = m

    mx_ref[...] = jnp.maximum(mx_ref[...], m)


def _tc_stage0(x, W):
    n, d = x.shape
    grid = n // _BLK
    return pl.pallas_call(
        _tc_stage0_body,
        grid=(grid,),
        in_specs=[pl.BlockSpec((_BLK, d), lambda i: (i, 0)),
                  pl.BlockSpec((d, d), lambda i: (0, 0))],
        out_specs=[pl.BlockSpec((_BLK, d), lambda i: (i, 0)),
                   pl.BlockSpec((1, d), lambda i: (0, 0))],
        out_shape=[jax.ShapeDtypeStruct((n, d), jnp.float32),
                   jax.ShapeDtypeStruct((1, d), jnp.float32)],
    )(x, W)


def _tc_mid_body(a_ref, sp_ref, b_ref, w_ref, hw_ref, mx_ref):
    i = pl.program_id(0)
    nq = a_ref.shape[0]
    a = jnp.concatenate([a_ref[q] for q in range(nq)], axis=-1)
    s = jnp.sum(sp_ref[...], axis=1)
    h = a + s[:, None] * b_ref[...]
    h = jnp.where(h >= 0, h, 0.01 * h)
    hw_ref[...] = jnp.dot(h, w_ref[...], precision=lax.Precision.HIGHEST)
    m = jnp.max(h, axis=0, keepdims=True)

    @pl.when(i == 0)
    def _():
        mx_ref[...] = m

    mx_ref[...] = jnp.maximum(mx_ref[...], m)


def _tc_mid(a4, sparts, b, W, n):
    d = W.shape[0]
    nq = a4.shape[0]
    sparts = sparts.T
    npart = sparts.shape[1]
    grid = n // _BLK
    return pl.pallas_call(
        _tc_mid_body,
        grid=(grid,),
        in_specs=[pl.BlockSpec((nq, _BLK, _QCOL), lambda i: (0, i, 0)),
                  pl.BlockSpec((_BLK, npart), lambda i: (i, 0)),
                  pl.BlockSpec((1, d), lambda i: (0, 0)),
                  pl.BlockSpec((d, d), lambda i: (0, 0))],
        out_specs=[pl.BlockSpec((_BLK, d), lambda i: (i, 0)),
                   pl.BlockSpec((1, d), lambda i: (0, 0))],
        out_shape=[jax.ShapeDtypeStruct((n, d), jnp.float32),
                   jax.ShapeDtypeStruct((1, d), jnp.float32)],
    )(a4, sparts, b.reshape(1, d), W)


def _tc_final_body(a_ref, sp_ref, b_ref, m0_ref, m1_ref, out_ref):
    i = pl.program_id(0)
    ng = pl.num_programs(0)
    nq = a_ref.shape[0]
    a = jnp.concatenate([a_ref[q] for q in range(nq)], axis=-1)
    s = jnp.sum(sp_ref[...], axis=1)
    h = a + s[:, None] * b_ref[...]
    h = jnp.where(h >= 0, h, 0.01 * h)
    m = jnp.max(h, axis=0, keepdims=True)

    @pl.when(i == 0)
    def _():
        out_ref[...] = m

    out_ref[...] = jnp.maximum(out_ref[...], m)

    @pl.when(i == ng - 1)
    def _():
        out_ref[...] = (m0_ref[...] + m1_ref[...] + out_ref[...]) / 3.0


def _tc_final(a4, sparts, b, m0, m1, n):
    d = b.shape[0]
    nq = a4.shape[0]
    sparts = sparts.T
    npart = sparts.shape[1]
    grid = n // _BLK
    return pl.pallas_call(
        _tc_final_body,
        grid=(grid,),
        in_specs=[pl.BlockSpec((nq, _BLK, _QCOL), lambda i: (0, i, 0)),
                  pl.BlockSpec((_BLK, npart), lambda i: (i, 0)),
                  pl.BlockSpec((1, d), lambda i: (0, 0)),
                  pl.BlockSpec((1, d), lambda i: (0, 0)),
                  pl.BlockSpec((1, d), lambda i: (0, 0))],
        out_specs=pl.BlockSpec((1, d), lambda i: (0, 0)),
        out_shape=jax.ShapeDtypeStruct((1, d), jnp.float32),
    )(a4, sparts, b.reshape(1, d), m0, m1)


def kernel(x, edge_index, node_norm, edge_norm, W, b):
    n, d = x.shape
    e = edge_index.shape[1]
    src = edge_index[0]
    dst = edge_index[1]

    quant = math.lcm(_NS * _CHUNK * 2 * _GRP, _NC * _NS * 16)
    e_pad = ((e + quant - 1) // quant) * quant
    pad = e_pad - e
    src_p = jnp.pad(src, (0, pad))
    dst_p = jnp.pad(dst, (0, pad))
    en_p = jnp.pad(edge_norm, (0, pad))

    n_pad = ((n + _NS * 8 - 1) // (_NS * 8)) * (_NS * 8)

    norm, sparts = _sc_edge_norms(node_norm, src_p, dst_p, en_p, n_pad)

    nchunk = e_pad // (_NS * _CHUNK)
    src_t = src_p.reshape(_NS, nchunk, _CHUNK)
    dst_t = dst_p.reshape(_NS, nchunk, _CHUNK)
    norm_t = norm.reshape(_NS, nchunk, _CHUNK)
    zrows = jnp.zeros((n_pad // _NS, _QCOL), jnp.float32)

    hw, m0 = _tc_stage0(x, W)
    a4 = _sc_aggregate(hw.reshape(4 * n, _QCOL), src_t, dst_t, norm_t,
                       zrows, n_pad)
    hw, m1 = _tc_mid(a4, sparts, b, W, n)
    a4 = _sc_aggregate(hw.reshape(4 * n, _QCOL), src_t, dst_t, norm_t,
                       zrows, n_pad)
    return _tc_final(a4, sparts, b, m0, m1, n)
